# Initial kernel scaffold; baseline (speedup 1.0000x reference)
import jax
import jax.numpy as jnp
from jax.experimental import pallas as pl


def kernel(x, edge_index, batch, W_pre1, b_pre1, g_pre1, be_pre1, W_pre2, b_pre2, W_proj, b_proj, W_post1, b_post1, g_post1, be_post1, W_post2, b_post2, g_post2, be_post2, W_post3, b_post3, W_h1, b_h1, W_h2, b_h2, W_h3, b_h3):
    raise NotImplementedError("write your pallas kernel here")


# speedup vs baseline: 28.8117x; 28.8117x over previous
import functools

import jax
import jax.numpy as jnp
from jax import lax
from jax.experimental import pallas as pl
from jax.experimental.pallas import tpu as pltpu
from jax.experimental.pallas import tpu_sc as plsc

N = 10000
E = 320000
D = 128
H = 128
OUT = 10
K = 5
KP = 8
CH = 80
NROWS = E // CH
NWORK = 32
RPW = NROWS // NWORK
NPAD = 10240
NPT = NPAD // 16
EPS = 1e-5


@functools.lru_cache(maxsize=None)
def _make_sc_agg(W):
    mesh = plsc.VectorSubcoreMesh(
        core_axis_name="c", subcore_axis_name="s", num_cores=2,
        num_subcores=16)

    @functools.partial(
        pl.kernel,
        out_type=jax.ShapeDtypeStruct((2, NPAD, W), jnp.float32),
        mesh=mesh,
        scratch_types=[
            pltpu.VMEM((RPW, CH), jnp.int32),
            pltpu.VMEM((RPW, CH), jnp.int32),
            pltpu.VMEM((CH, W), jnp.float32),
            pltpu.VMEM_SHARED((NPAD, W), jnp.float32),
            pltpu.SemaphoreType.DMA,
        ],
        compiler_params=pltpu.CompilerParams(use_tc_tiling_on_sc=False),
    )
    def sc_agg(tab_hbm, src_hbm, dst_hbm, zero_hbm, out_hbm,
               src_v, dst_v, rows_v, acc_sh, sem):
        c = lax.axis_index("c")
        s = lax.axis_index("s")
        wid = c * 16 + s

        pltpu.sync_copy(zero_hbm.at[pl.ds(s * NPT, NPT)],
                        acc_sh.at[pl.ds(s * NPT, NPT)])

        pltpu.sync_copy(src_hbm.at[wid], src_v)
        pltpu.sync_copy(dst_hbm.at[wid], dst_v)
        plsc.subcore_barrier()

        def body(j, carry):
            pltpu.async_copy(tab_hbm.at[src_v.at[j]], rows_v, sem).wait()
            pltpu.sync_copy(rows_v, acc_sh.at[dst_v.at[j]], add=True)
            return carry

        lax.fori_loop(0, RPW, body, 0)
        plsc.subcore_barrier()

        pltpu.sync_copy(acc_sh.at[pl.ds(s * NPT, NPT)],
                        out_hbm.at[c, pl.ds(s * NPT, NPT)])

    return sc_agg


def _dinv_from_degp(degp_ref):
    deg = degp_ref[0][:N, 0:1] + degp_ref[1][:N, 0:1] + 1.0
    return lax.rsqrt(deg)


def _tc1_body(x_ref, w_ref, degp_ref, z_ref):
    dinv = _dinv_from_degp(degp_ref)
    y = jnp.dot(x_ref[...], w_ref[...], preferred_element_type=jnp.float32)
    z_ref[...] = y * dinv


def _tc2_body(p_ref, z_ref, degp_ref, b1_ref, g1_ref, be1_ref, w2_ref,
              z2_ref):
    dinv = _dinv_from_degp(degp_ref)
    out1 = (p_ref[0][:N] + p_ref[1][:N] + z_ref[...]) * dinv + b1_ref[...]
    mu = jnp.mean(out1, axis=0, keepdims=True)
    var = jnp.mean((out1 - mu) ** 2, axis=0, keepdims=True)
    h1 = (out1 - mu) / jnp.sqrt(var + EPS) * g1_ref[...] + be1_ref[...]
    h1 = jax.nn.gelu(h1)
    y2 = jnp.dot(h1, w2_ref[...], preferred_element_type=jnp.float32)
    z2_ref[...] = y2 * dinv


def _tc3_body(p_ref, z2_ref, degp_ref, b2_ref, oh_ref, sums_ref, cnt_ref):
    dinv = _dinv_from_degp(degp_ref)
    h = (p_ref[0][:N] + p_ref[1][:N] + z2_ref[...]) * dinv + b2_ref[...]

    ones_col = jnp.ones((N, 1), jnp.float32)
    cent = jnp.concatenate([h[0:K], jnp.zeros((KP - K, H), jnp.float32)], 0)
    oh = None
    sums = None
    cnt = None
    for _ in range(10):
        best_d = jnp.sum((h - cent[0:1]) ** 2, axis=1, keepdims=True)
        best_k = jnp.zeros((N, 1), jnp.int32)
        for k in range(1, K):
            dk = jnp.sum((h - cent[k:k + 1]) ** 2, axis=1, keepdims=True)
            upd = dk < best_d
            best_d = jnp.where(upd, dk, best_d)
            best_k = jnp.where(upd, k, best_k)
        oh = (best_k == lax.broadcasted_iota(jnp.int32, (N, KP), 1)
              ).astype(jnp.float32)
        sums = lax.dot_general(oh, h, (((0,), (0,)), ((), ())),
                               preferred_element_type=jnp.float32)
        cnt = lax.dot_general(oh, ones_col, (((0,), (0,)), ((), ())),
                              preferred_element_type=jnp.float32)
        cent = sums / jnp.maximum(cnt, 1.0)

    oh_ref[...] = oh
    sums_ref[...] = sums
    cnt_ref[...] = jnp.broadcast_to(cnt, (KP, H))


def _tc4_body(sums_ref, cnt_ref, pp_ref, oh_ref, wproj_ref, bproj_ref,
              w1_ref, b1_ref, g1_ref, be1_ref, w2_ref, b2_ref, g2_ref,
              be2_ref, w3_ref, b3_ref, wh1_ref, bh1_ref, wh2_ref, bh2_ref,
              wh3_ref, bh3_ref, out_ref):
    cnt = cnt_ref[...][:, 0:1]
    sums = sums_ref[...]
    xc = sums / jnp.maximum(cnt, 1.0)
    cmaskf = (cnt > 0).astype(jnp.float32)
    nc = jnp.sum(cmaskf)

    pagg = pp_ref[0][:N] + pp_ref[1][:N]
    G = lax.dot_general(oh_ref[...], pagg, (((0,), (0,)), ((), ())),
                        preferred_element_type=jnp.float32)
    eye = (lax.broadcasted_iota(jnp.int32, (KP, KP), 0)
           == lax.broadcasted_iota(jnp.int32, (KP, KP), 1))
    G = jnp.where(eye, 0.0, G)
    indeg = jnp.sum(G, axis=1, keepdims=True)
    dinv_c = lax.rsqrt(indeg + 1.0)

    def cconv(v, w, b):
        y = jnp.dot(v, w, preferred_element_type=jnp.float32)
        t = jnp.dot(G, dinv_c * y, preferred_element_type=jnp.float32)
        return dinv_c * t + (dinv_c * dinv_c) * y + b

    def bn_mask(v, g, b):
        mu = jnp.sum(v * cmaskf, axis=0, keepdims=True) / nc
        var = jnp.sum(((v - mu) ** 2) * cmaskf, axis=0, keepdims=True) / nc
        return (v - mu) / jnp.sqrt(var + EPS) * g + b

    xc = jnp.dot(xc, wproj_ref[...], preferred_element_type=jnp.float32) \
        + bproj_ref[...]
    h2 = cconv(xc, w1_ref[...], b1_ref[...])
    h2 = jax.nn.gelu(bn_mask(h2, g1_ref[...], be1_ref[...]))
    h2 = cconv(h2, w2_ref[...], b2_ref[...])
    h2 = jax.nn.gelu(bn_mask(h2, g2_ref[...], be2_ref[...]))
    h2 = cconv(h2, w3_ref[...], b3_ref[...])

    pooled = jnp.sum(h2 * cmaskf, axis=0, keepdims=True) \
        / jnp.maximum(nc, 1.0)
    h3 = jax.nn.gelu(jnp.dot(pooled, wh1_ref[...],
                             preferred_element_type=jnp.float32)
                     + bh1_ref[...])
    h3 = jax.nn.gelu(jnp.dot(h3, wh2_ref[...],
                             preferred_element_type=jnp.float32)
                     + bh2_ref[...])
    out_ref[...] = jnp.dot(h3, wh3_ref[...],
                           preferred_element_type=jnp.float32) + bh3_ref[...]


def _f32(shape):
    return jax.ShapeDtypeStruct(shape, jnp.float32)


_TC_PARAMS = pltpu.CompilerParams(vmem_limit_bytes=100 * 1024 * 1024)

_tc1 = pl.pallas_call(_tc1_body, out_shape=_f32((N, H)),
                      compiler_params=_TC_PARAMS)
_tc2 = pl.pallas_call(_tc2_body, out_shape=_f32((N, H)),
                      compiler_params=_TC_PARAMS)
_tc3 = pl.pallas_call(
    _tc3_body, out_shape=(_f32((N, KP)), _f32((KP, H)), _f32((KP, H))),
    compiler_params=_TC_PARAMS)
_tc4 = pl.pallas_call(_tc4_body, out_shape=_f32((1, H)),
                      compiler_params=_TC_PARAMS)


def kernel(x, edge_index, batch, W_pre1, b_pre1, g_pre1, be_pre1, W_pre2,
           b_pre2, W_proj, b_proj, W_post1, b_post1, g_post1, be_post1,
           W_post2, b_post2, g_post2, be_post2, W_post3, b_post3, W_h1,
           b_h1, W_h2, b_h2, W_h3, b_h3):
    src3d = edge_index[0].reshape(NWORK, RPW, CH)
    dst3d = edge_index[1].reshape(NWORK, RPW, CH)
    ones8 = jnp.ones((N, KP), jnp.float32)
    zeros8 = jnp.zeros((NPAD, KP), jnp.float32)
    zeros128 = jnp.zeros((NPAD, H), jnp.float32)

    row = lambda v: v.reshape(1, -1)

    sc_agg8 = _make_sc_agg(KP)
    sc_agg128 = _make_sc_agg(H)

    degp = sc_agg8(ones8, src3d, dst3d, zeros8)
    z1 = _tc1(x, W_pre1, degp)
    p1 = sc_agg128(z1, src3d, dst3d, zeros128)
    z2 = _tc2(p1, z1, degp, row(b_pre1), row(g_pre1), row(be_pre1), W_pre2)
    p2 = sc_agg128(z2, src3d, dst3d, zeros128)
    oh, sums, cntb = _tc3(p2, z2, degp, row(b_pre2))
    pp = sc_agg8(oh, src3d, dst3d, zeros8)

    wh3p = jnp.pad(W_h3, ((0, 0), (0, H - OUT)))
    bh3p = jnp.pad(b_h3, (0, H - OUT))
    outp = _tc4(sums, cntb, pp, oh, W_proj, row(b_proj), W_post1,
                row(b_post1), row(g_post1), row(be_post1), W_post2,
                row(b_post2), row(g_post2), row(be_post2), W_post3,
                row(b_post3), W_h1, row(b_h1), W_h2, row(b_h2), wh3p,
                row(bh3p))
    return outp[:, :OUT]


# speedup vs baseline: 43.2178x; 1.5000x over previous
import functools

import jax
import jax.numpy as jnp
from jax import lax
from jax.experimental import pallas as pl
from jax.experimental.pallas import tpu as pltpu
from jax.experimental.pallas import tpu_sc as plsc

N = 10000
E = 320000
D = 128
H = 128
OUT = 10
K = 5
KP = 8
CH = 100
NROWS = E // CH
NWORK = 32
RPW = NROWS // NWORK
NPAD = 10240
NPT = NPAD // 16
EPS = 1e-5


@functools.lru_cache(maxsize=None)
def _make_sc_agg(W):
    mesh = plsc.VectorSubcoreMesh(
        core_axis_name="c", subcore_axis_name="s", num_cores=2,
        num_subcores=16)

    @functools.partial(
        pl.kernel,
        out_type=jax.ShapeDtypeStruct((2, NPAD, W), jnp.float32),
        mesh=mesh,
        scratch_types=[
            pltpu.VMEM((RPW, CH), jnp.int32),
            pltpu.VMEM((RPW, CH), jnp.int32),
            pltpu.VMEM((CH, W), jnp.float32),
            pltpu.VMEM((CH, W), jnp.float32),
            pltpu.VMEM_SHARED((NPAD, W), jnp.float32),
            pltpu.SemaphoreType.DMA,
            pltpu.SemaphoreType.DMA,
        ],
        compiler_params=pltpu.CompilerParams(use_tc_tiling_on_sc=False),
    )
    def sc_agg(tab_hbm, src_hbm, dst_hbm, zero_hbm, out_hbm,
               src_v, dst_v, rows0_v, rows1_v, acc_sh, sem0, sem1):
        c = lax.axis_index("c")
        s = lax.axis_index("s")
        wid = c * 16 + s
        rows = (rows0_v, rows1_v)
        sems = (sem0, sem1)

        pltpu.sync_copy(zero_hbm.at[pl.ds(s * NPT, NPT)],
                        acc_sh.at[pl.ds(s * NPT, NPT)])

        pltpu.sync_copy(src_hbm.at[wid], src_v)
        pltpu.sync_copy(dst_hbm.at[wid], dst_v)
        plsc.subcore_barrier()

        for b in range(2):
            pltpu.async_copy(tab_hbm.at[src_v.at[b]], rows[b], sems[b])

        def body(i, carry):
            for b in range(2):
                j = 2 * i + b
                pltpu.make_async_copy(tab_hbm.at[src_v.at[j]], rows[b],
                                      sems[b]).wait()
                pltpu.sync_copy(rows[b], acc_sh.at[dst_v.at[j]], add=True)
                pltpu.async_copy(tab_hbm.at[src_v.at[j + 2]], rows[b],
                                 sems[b])
            return carry

        lax.fori_loop(0, RPW // 2 - 1, body, 0)
        for b in range(2):
            j = RPW - 2 + b
            pltpu.make_async_copy(tab_hbm.at[src_v.at[j]], rows[b],
                                  sems[b]).wait()
            pltpu.sync_copy(rows[b], acc_sh.at[dst_v.at[j]], add=True)
        plsc.subcore_barrier()

        pltpu.sync_copy(acc_sh.at[pl.ds(s * NPT, NPT)],
                        out_hbm.at[c, pl.ds(s * NPT, NPT)])

    return sc_agg


def _gelu(x):
    return jax.nn.gelu(x)


def _dinv_from_degp(degp_ref):
    deg = degp_ref[0][:N, 0:1] + degp_ref[1][:N, 0:1] + 1.0
    return 1.0 / jnp.sqrt(deg)


def _tc1_body(x_ref, w_ref, degp_ref, z_ref):
    dinv = _dinv_from_degp(degp_ref)
    y = jnp.dot(x_ref[...], w_ref[...], preferred_element_type=jnp.float32)
    z_ref[...] = y * dinv


def _tc2_body(p_ref, z_ref, degp_ref, b1_ref, g1_ref, be1_ref, w2_ref,
              z2_ref):
    dinv = _dinv_from_degp(degp_ref)
    out1 = (p_ref[0][:N] + p_ref[1][:N] + z_ref[...]) * dinv + b1_ref[...]
    mu = jnp.mean(out1, axis=0, keepdims=True)
    var = jnp.mean((out1 - mu) ** 2, axis=0, keepdims=True)
    h1 = (out1 - mu) / jnp.sqrt(var + EPS) * g1_ref[...] + be1_ref[...]
    h1 = _gelu(h1)
    y2 = jnp.dot(h1, w2_ref[...], preferred_element_type=jnp.float32)
    z2_ref[...] = y2 * dinv


def _tc3_body(p_ref, z2_ref, degp_ref, b2_ref, oh_ref, sums_ref, cnt_ref):
    dinv = _dinv_from_degp(degp_ref)
    h = (p_ref[0][:N] + p_ref[1][:N] + z2_ref[...]) * dinv + b2_ref[...]

    ones_col = jnp.ones((N, 1), jnp.float32)
    cent = jnp.concatenate([h[0:K], jnp.zeros((KP - K, H), jnp.float32)], 0)
    oh = None
    sums = None
    cnt = None
    for _ in range(10):
        best_d = jnp.sum((h - cent[0:1]) ** 2, axis=1, keepdims=True)
        best_k = jnp.zeros((N, 1), jnp.int32)
        for k in range(1, K):
            dk = jnp.sum((h - cent[k:k + 1]) ** 2, axis=1, keepdims=True)
            upd = dk < best_d
            best_d = jnp.where(upd, dk, best_d)
            best_k = jnp.where(upd, k, best_k)
        oh = (best_k == lax.broadcasted_iota(jnp.int32, (N, KP), 1)
              ).astype(jnp.float32)
        sums = lax.dot_general(oh, h, (((0,), (0,)), ((), ())),
                               preferred_element_type=jnp.float32,
                               precision=lax.Precision.HIGHEST)
        cnt = lax.dot_general(oh, ones_col, (((0,), (0,)), ((), ())),
                              preferred_element_type=jnp.float32,
                              precision=lax.Precision.HIGHEST)
        cent = sums / jnp.maximum(cnt, 1.0)

    oh_ref[...] = oh
    sums_ref[...] = sums
    cnt_ref[...] = jnp.broadcast_to(cnt, (KP, H))


def _tc4_body(sums_ref, cnt_ref, pp_ref, oh_ref, wproj_ref, bproj_ref,
              w1_ref, b1_ref, g1_ref, be1_ref, w2_ref, b2_ref, g2_ref,
              be2_ref, w3_ref, b3_ref, wh1_ref, bh1_ref, wh2_ref, bh2_ref,
              wh3_ref, bh3_ref, out_ref):
    cnt = cnt_ref[...][:, 0:1]
    sums = sums_ref[...]
    xc = sums / jnp.maximum(cnt, 1.0)
    cmaskf = (cnt > 0).astype(jnp.float32)
    nc = jnp.sum(cmaskf)

    pagg = pp_ref[0][:N] + pp_ref[1][:N]
    G = lax.dot_general(oh_ref[...], pagg, (((0,), (0,)), ((), ())),
                        preferred_element_type=jnp.float32)
    eye = (lax.broadcasted_iota(jnp.int32, (KP, KP), 0)
           == lax.broadcasted_iota(jnp.int32, (KP, KP), 1))
    G = jnp.where(eye, 0.0, G)
    indeg = jnp.sum(G, axis=1, keepdims=True)
    dinv_c = 1.0 / jnp.sqrt(indeg + 1.0)

    def cconv(v, w, b):
        y = jnp.dot(v, w, preferred_element_type=jnp.float32)
        t = jnp.dot(G, dinv_c * y, preferred_element_type=jnp.float32,
                    precision=lax.Precision.HIGHEST)
        return dinv_c * t + (dinv_c * dinv_c) * y + b

    def bn_mask(v, g, b):
        mu = jnp.sum(v * cmaskf, axis=0, keepdims=True) / nc
        var = jnp.sum(((v - mu) ** 2) * cmaskf, axis=0, keepdims=True) / nc
        return (v - mu) / jnp.sqrt(var + EPS) * g + b

    xc = jnp.dot(xc, wproj_ref[...], preferred_element_type=jnp.float32) \
        + bproj_ref[...]
    h2 = cconv(xc, w1_ref[...], b1_ref[...])
    h2 = _gelu(bn_mask(h2, g1_ref[...], be1_ref[...]))
    h2 = cconv(h2, w2_ref[...], b2_ref[...])
    h2 = _gelu(bn_mask(h2, g2_ref[...], be2_ref[...]))
    h2 = cconv(h2, w3_ref[...], b3_ref[...])

    pooled = jnp.sum(h2 * cmaskf, axis=0, keepdims=True) \
        / jnp.maximum(nc, 1.0)
    h3 = _gelu(jnp.dot(pooled, wh1_ref[...],
                       preferred_element_type=jnp.float32) + bh1_ref[...])
    h3 = _gelu(jnp.dot(h3, wh2_ref[...],
                       preferred_element_type=jnp.float32) + bh2_ref[...])
    out_ref[...] = jnp.dot(h3, wh3_ref[...],
                           preferred_element_type=jnp.float32) + bh3_ref[...]


def _f32(shape):
    return jax.ShapeDtypeStruct(shape, jnp.float32)


_TC_PARAMS = pltpu.CompilerParams(vmem_limit_bytes=100 * 1024 * 1024)

_tc1 = pl.pallas_call(_tc1_body, out_shape=_f32((N, H)),
                      compiler_params=_TC_PARAMS)
_tc2 = pl.pallas_call(_tc2_body, out_shape=_f32((N, H)),
                      compiler_params=_TC_PARAMS)
_tc3 = pl.pallas_call(
    _tc3_body, out_shape=(_f32((N, KP)), _f32((KP, H)), _f32((KP, H))),
    compiler_params=_TC_PARAMS)
_tc4 = pl.pallas_call(_tc4_body, out_shape=_f32((1, H)),
                      compiler_params=_TC_PARAMS)


def kernel(x, edge_index, batch, W_pre1, b_pre1, g_pre1, be_pre1, W_pre2,
           b_pre2, W_proj, b_proj, W_post1, b_post1, g_post1, be_post1,
           W_post2, b_post2, g_post2, be_post2, W_post3, b_post3, W_h1,
           b_h1, W_h2, b_h2, W_h3, b_h3):
    src3d = edge_index[0].reshape(NWORK, RPW, CH)
    dst3d = edge_index[1].reshape(NWORK, RPW, CH)
    ones8 = jnp.ones((N, KP), jnp.float32)
    zeros8 = jnp.zeros((NPAD, KP), jnp.float32)
    zeros128 = jnp.zeros((NPAD, H), jnp.float32)

    row = lambda v: v.reshape(1, -1)

    sc_agg8 = _make_sc_agg(KP)
    sc_agg128 = _make_sc_agg(H)

    degp = sc_agg8(ones8, src3d, dst3d, zeros8)
    z1 = _tc1(x, W_pre1, degp)
    p1 = sc_agg128(z1, src3d, dst3d, zeros128)
    z2 = _tc2(p1, z1, degp, row(b_pre1), row(g_pre1), row(be_pre1), W_pre2)
    p2 = sc_agg128(z2, src3d, dst3d, zeros128)
    oh, sums, cntb = _tc3(p2, z2, degp, row(b_pre2))
    pp = sc_agg8(oh, src3d, dst3d, zeros8)

    wh3p = jnp.pad(W_h3, ((0, 0), (0, H - OUT)))
    bh3p = jnp.pad(b_h3, (0, H - OUT))
    outp = _tc4(sums, cntb, pp, oh, W_proj, row(b_proj), W_post1,
                row(b_post1), row(g_post1), row(be_post1), W_post2,
                row(b_post2), row(g_post2), row(be_post2), W_post3,
                row(b_post3), W_h1, row(b_h1), W_h2, row(b_h2), wh3p,
                row(bh3p))
    return outp[:, :OUT]
